# baseline (speedup 1.0000x reference)
import functools

import jax
import jax.numpy as jnp
from jax.experimental import pallas as pl
from jax.experimental.pallas import tpu as pltpu

_D = 64
_T = 10
_N = 100000

_FBLK = 6272
_FROWS = _FBLK // 128
_NBLKS = 16
_NSTREAMS = 2
_KSTEPS = _NBLKS // _NSTREAMS


def _store_block(e_ref, m_ref, s_ref, l, jblk, need_mask):
    l2 = l.reshape(_FROWS, 128)
    if need_mask:
        rows = jax.lax.broadcasted_iota(jnp.int32, (_FROWS, 128), 0)
        lanes = jax.lax.broadcasted_iota(jnp.int32, (_FROWS, 128), 1)
        gcol = rows * 128 + lanes + jblk * _FBLK
        l2 = jnp.where(gcol < _N, l2, -jnp.inf)
    m = jnp.max(l2)
    e2 = jnp.exp(l2 - m)
    e_ref[pl.ds(jblk * _FROWS, _FROWS), :] = e2
    m_ref[:, pl.ds(jblk * 128, 128)] = jnp.full((1, 128), m, jnp.float32)
    s_ref[:, pl.ds(jblk * 128, 128)] = jnp.full((1, 128), jnp.sum(e2),
                                                jnp.float32)


def _tc_fused(h_ref, cls_ref, r0_ref, r1_ref,
              pred_ref, attn_ref, e_ref, m_ref, s_ref):
    i = pl.program_id(0)

    @pl.when(i < _KSTEPS)
    def _phase1():
        h = h_ref[:, :]
        for s, rref in enumerate((r0_ref, r1_ref)):
            l = jnp.dot(h, rref[:, :], preferred_element_type=jnp.float32)
            jblk = s * _KSTEPS + i
            _store_block(e_ref, m_ref, s_ref, l, jblk, s == _NSTREAMS - 1)

    @pl.when(i == 0)
    def _pred():
        pred_ref[:, :] = jax.lax.dot_general(
            h_ref[:, :], cls_ref[:, :], (((1,), (1,)), ((), ())),
            preferred_element_type=jnp.float32)

    @pl.when(i == _KSTEPS)
    def _finale():
        mrow = m_ref[:, :]
        srow = s_ref[:, :]
        big = jnp.max(mrow)
        w = srow * jnp.exp(mrow - big)
        total = jnp.sum(w) * (1.0 / 128.0)
        scales = jnp.exp(mrow - big) * (1.0 / total)
        for j in range(_NBLKS):
            sv = jnp.max(scales[:, j * 128:(j + 1) * 128])
            e2 = e_ref[pl.ds(j * _FROWS, _FROWS), :]
            seg = (e2 * sv).reshape(1, _FBLK)
            width = min(_FBLK, _N - j * _FBLK)
            attn_ref[:, pl.ds(j * _FBLK, width)] = seg[:, :width]


@jax.jit
def kernel(hidden_state, theta_classification, theta_rank):
    h = hidden_state.reshape(1, _D)
    cls_t = theta_classification.T
    pred, attn = pl.pallas_call(
        _tc_fused,
        grid=(_KSTEPS + 1,),
        in_specs=[
            pl.BlockSpec((1, _D), lambda i: (0, 0)),
            pl.BlockSpec((_T, _D), lambda i: (0, 0)),
        ] + [
            pl.BlockSpec(
                (_D, _FBLK),
                functools.partial(
                    lambda s, i: (0, s * _KSTEPS + jnp.minimum(i, _KSTEPS - 1)),
                    s))
            for s in range(_NSTREAMS)
        ],
        out_specs=[
            pl.BlockSpec((1, _T), lambda i: (0, 0)),
            pl.BlockSpec((1, _N), lambda i: (0, 0)),
        ],
        out_shape=[
            jax.ShapeDtypeStruct((1, _T), jnp.float32),
            jax.ShapeDtypeStruct((1, _N), jnp.float32),
        ],
        scratch_shapes=[
            pltpu.VMEM((_NBLKS * _FROWS, 128), jnp.float32),
            pltpu.VMEM((1, _NBLKS * 128), jnp.float32),
            pltpu.VMEM((1, _NBLKS * 128), jnp.float32),
        ],
    )(h, cls_t, theta_rank, theta_rank)
    return (pred, attn)


# speedup vs baseline: 1.2131x; 1.2131x over previous
import functools

import jax
import jax.numpy as jnp
from jax.experimental import pallas as pl
from jax.experimental.pallas import tpu as pltpu

_D = 64
_T = 10
_N = 100000

_FBLK = 12544
_FROWS = _FBLK // 128
_NBLKS = 8
_NSTREAMS = 2
_KSTEPS = _NBLKS // _NSTREAMS


def _store_block(e_ref, m_ref, s_ref, l, jblk, need_mask):
    l2 = l.reshape(_FROWS, 128)
    if need_mask:
        rows = jax.lax.broadcasted_iota(jnp.int32, (_FROWS, 128), 0)
        lanes = jax.lax.broadcasted_iota(jnp.int32, (_FROWS, 128), 1)
        gcol = rows * 128 + lanes + jblk * _FBLK
        l2 = jnp.where(gcol < _N, l2, -jnp.inf)
    m = jnp.max(l2)
    e2 = jnp.exp(l2 - m)
    e_ref[pl.ds(jblk * _FROWS, _FROWS), :] = e2
    m_ref[:, pl.ds(jblk * 128, 128)] = jnp.full((1, 128), m, jnp.float32)
    s_ref[:, pl.ds(jblk * 128, 128)] = jnp.full((1, 128), jnp.sum(e2),
                                                jnp.float32)


def _tc_fused(h_ref, cls_ref, r0_ref, r1_ref,
              pred_ref, attn_ref, e_ref, m_ref, s_ref):
    i = pl.program_id(0)

    @pl.when(i < _KSTEPS)
    def _phase1():
        h = h_ref[:, :]
        for s, rref in enumerate((r0_ref, r1_ref)):
            l = jnp.dot(h, rref[:, :], preferred_element_type=jnp.float32)
            jblk = s * _KSTEPS + i
            _store_block(e_ref, m_ref, s_ref, l, jblk, s == _NSTREAMS - 1)

    @pl.when(i == 0)
    def _pred():
        pred_ref[:, :] = jax.lax.dot_general(
            h_ref[:, :], cls_ref[:, :], (((1,), (1,)), ((), ())),
            preferred_element_type=jnp.float32)

    @pl.when(i == _KSTEPS)
    def _finale():
        mrow = m_ref[:, :]
        srow = s_ref[:, :]
        big = jnp.max(mrow)
        w = srow * jnp.exp(mrow - big)
        total = jnp.sum(w) * (1.0 / 128.0)
        scales = jnp.exp(mrow - big) * (1.0 / total)
        for j in range(_NBLKS):
            sv = jnp.max(scales[:, j * 128:(j + 1) * 128])
            e2 = e_ref[pl.ds(j * _FROWS, _FROWS), :]
            seg = (e2 * sv).reshape(1, _FBLK)
            width = min(_FBLK, _N - j * _FBLK)
            attn_ref[:, pl.ds(j * _FBLK, width)] = seg[:, :width]


@jax.jit
def kernel(hidden_state, theta_classification, theta_rank):
    h = hidden_state.reshape(1, _D)
    cls_t = theta_classification.T
    pred, attn = pl.pallas_call(
        _tc_fused,
        grid=(_KSTEPS + 1,),
        in_specs=[
            pl.BlockSpec((1, _D), lambda i: (0, 0)),
            pl.BlockSpec((_T, _D), lambda i: (0, 0)),
        ] + [
            pl.BlockSpec(
                (_D, _FBLK),
                functools.partial(
                    lambda s, i: (0, s * _KSTEPS + jnp.minimum(i, _KSTEPS - 1)),
                    s))
            for s in range(_NSTREAMS)
        ],
        out_specs=[
            pl.BlockSpec((1, _T), lambda i: (0, 0)),
            pl.BlockSpec((1, _N), lambda i: (0, 0)),
        ],
        out_shape=[
            jax.ShapeDtypeStruct((1, _T), jnp.float32),
            jax.ShapeDtypeStruct((1, _N), jnp.float32),
        ],
        scratch_shapes=[
            pltpu.VMEM((_NBLKS * _FROWS, 128), jnp.float32),
            pltpu.VMEM((1, _NBLKS * 128), jnp.float32),
            pltpu.VMEM((1, _NBLKS * 128), jnp.float32),
        ],
    )(h, cls_t, theta_rank, theta_rank)
    return (pred, attn)
